# baseline (speedup 1.0000x reference)
import functools

import jax
import jax.numpy as jnp
from jax import lax
from jax.experimental import pallas as pl
from jax.experimental.pallas import tpu as pltpu
from jax.experimental.pallas import tpu_sc as plsc

_CHUNK = 80


def _make_sc_gather(n_rows, width, n_workers):
    rows_per_w = n_rows // n_workers
    n_chunks = rows_per_w // _CHUNK
    assert rows_per_w % _CHUNK == 0

    mesh = plsc.VectorSubcoreMesh(core_axis_name="c", subcore_axis_name="s")

    @functools.partial(
        pl.kernel,
        mesh=mesh,
        out_type=jax.ShapeDtypeStruct((n_rows, width), jnp.float32),
        scratch_types=[
            pltpu.VMEM((rows_per_w,), jnp.int32),
            pltpu.VMEM((2, _CHUNK, width), jnp.float32),
            pltpu.SemaphoreType.DMA,
        ],
    )
    def sc_gather(idx_hbm, table_hbm, out_hbm, idx_v, rows_v, sem):
        nc = 2
        wid = lax.axis_index("s") * nc + lax.axis_index("c")
        base = wid * rows_per_w
        pltpu.sync_copy(idx_hbm.at[pl.ds(base, rows_per_w)], idx_v)

        def gather(j, buf):
            return pltpu.async_copy(
                table_hbm.at[idx_v.at[pl.ds(j * _CHUNK, _CHUNK)]],
                rows_v.at[buf],
                sem,
            )

        cp = gather(0, 0)
        for j in range(n_chunks):
            nxt = gather(j + 1, (j + 1) % 2) if j + 1 < n_chunks else None
            cp.wait()
            pltpu.sync_copy(
                rows_v.at[j % 2], out_hbm.at[pl.ds(base + j * _CHUNK, _CHUNK)]
            )
            cp = nxt

    return sc_gather


def _make_rnn_body(inner, seq):
    last_g, last_s = (seq - 1) // inner, (seq - 1) % inner

    def _rnn_steps(x_ref, m_ref, wih_ref, whh_ref, bias_ref, out_ref, hout_ref,
                   h_ref):
        g = pl.program_id(0)

        @pl.when(g == 0)
        def _():
            h_ref[...] = jnp.zeros_like(h_ref)

        hid = h_ref.shape[-1]
        h = h_ref[...]
        for s in range(inner):
            x128 = x_ref[s]
            x = jnp.where(m_ref[s].astype(jnp.int32) > 0, x128[:, hid:], x128[:, :hid])
            acc = jnp.dot(x, wih_ref[...], preferred_element_type=jnp.float32)
            acc += jnp.dot(h, whh_ref[...], preferred_element_type=jnp.float32)
            acc += bias_ref[...]
            h = jnp.tanh(acc)
            out_ref[:, s, :] = h
            if s == last_s:
                h_keep = h

        @pl.when(g == last_g)
        def _():
            hout_ref[...] = h_keep

        h_ref[...] = h

    return _rnn_steps


def kernel(sentences, emb_table, W_ih, W_hh, b_ih, b_hh):
    batch, seq = sentences.shape
    vocab, emb = emb_table.shape
    hid = W_hh.shape[0]
    n_rows = batch * seq

    table2 = emb_table.reshape(vocab // 2, 2 * emb)

    sent_tm = sentences.T.astype(jnp.int32)
    phys = (sent_tm >> 1).reshape(n_rows)
    mask8 = jnp.broadcast_to(
        (sent_tm & 1).astype(jnp.int8)[:, :, None], (seq, batch, emb)
    )

    xg = _make_sc_gather(n_rows, 2 * emb, 32)(phys, table2)
    xg = xg.reshape(seq, batch, 2 * emb)

    wih_t = W_ih.T
    whh_t = W_hh.T
    bias = (b_ih + b_hh).reshape(1, hid)

    inner = 8
    n_blocks = (seq + inner - 1) // inner
    final_output, h_last = pl.pallas_call(
        _make_rnn_body(inner, seq),
        grid=(n_blocks,),
        in_specs=[
            pl.BlockSpec((inner, batch, 2 * emb), lambda g: (g, 0, 0)),
            pl.BlockSpec((inner, batch, emb), lambda g: (g, 0, 0)),
            pl.BlockSpec((emb, hid), lambda g: (0, 0)),
            pl.BlockSpec((hid, hid), lambda g: (0, 0)),
            pl.BlockSpec((1, hid), lambda g: (0, 0)),
        ],
        out_specs=[
            pl.BlockSpec((batch, inner, hid), lambda g: (0, g, 0)),
            pl.BlockSpec((batch, hid), lambda g: (0, 0)),
        ],
        out_shape=[
            jax.ShapeDtypeStruct((batch, seq, hid), jnp.float32),
            jax.ShapeDtypeStruct((batch, hid), jnp.float32),
        ],
        scratch_shapes=[pltpu.VMEM((batch, hid), jnp.float32)],
    )(xg, mask8, wih_t, whh_t, bias)

    return final_output, h_last[None]


# speedup vs baseline: 1.1681x; 1.1681x over previous
import functools

import jax
import jax.numpy as jnp
from jax import lax
from jax.experimental import pallas as pl
from jax.experimental.pallas import tpu as pltpu
from jax.experimental.pallas import tpu_sc as plsc

_CHUNK = 80


def _make_sc_gather(n_rows, width, n_workers):
    rows_per_w = n_rows // n_workers
    n_chunks = rows_per_w // _CHUNK
    assert rows_per_w % _CHUNK == 0

    mesh = plsc.VectorSubcoreMesh(core_axis_name="c", subcore_axis_name="s")

    @functools.partial(
        pl.kernel,
        mesh=mesh,
        out_type=jax.ShapeDtypeStruct((n_rows, width), jnp.float32),
        scratch_types=[
            pltpu.VMEM((rows_per_w,), jnp.int32),
            pltpu.VMEM((2, _CHUNK, width), jnp.float32),
            pltpu.SemaphoreType.DMA,
        ],
    )
    def sc_gather(idx_hbm, table_hbm, out_hbm, idx_v, rows_v, sem):
        nc = 2
        wid = lax.axis_index("s") * nc + lax.axis_index("c")
        base = wid * rows_per_w
        pltpu.sync_copy(idx_hbm.at[pl.ds(base, rows_per_w)], idx_v)

        def gather(j, buf):
            return pltpu.async_copy(
                table_hbm.at[idx_v.at[pl.ds(j * _CHUNK, _CHUNK)]],
                rows_v.at[buf],
                sem,
            )

        cp = gather(0, 0)
        for j in range(n_chunks):
            nxt = gather(j + 1, (j + 1) % 2) if j + 1 < n_chunks else None
            cp.wait()
            pltpu.sync_copy(
                rows_v.at[j % 2], out_hbm.at[pl.ds(base + j * _CHUNK, _CHUNK)]
            )
            cp = nxt

    return sc_gather


def _make_rnn_body(inner, seq):
    last_g, last_s = (seq - 1) // inner, (seq - 1) % inner

    def _rnn_steps(x_ref, m_ref, wih_ref, whh_ref, bias_ref, out_ref, hout_ref,
                   h_ref):
        g = pl.program_id(0)

        @pl.when(g == 0)
        def _():
            h_ref[...] = jnp.zeros_like(h_ref)

        hid = h_ref.shape[-1]
        h = h_ref[...]
        for s in range(inner):
            x128 = x_ref[s]
            x = jnp.where(m_ref[s].astype(jnp.int32) > 0, x128[:, hid:], x128[:, :hid])
            acc = jnp.dot(x, wih_ref[...], preferred_element_type=jnp.float32)
            acc += jnp.dot(h, whh_ref[...], preferred_element_type=jnp.float32)
            acc += bias_ref[...]
            h = jnp.tanh(acc)
            out_ref[s] = h
            if s == last_s:
                h_keep = h

        @pl.when(g == last_g)
        def _():
            hout_ref[...] = h_keep

        h_ref[...] = h

    return _rnn_steps


def kernel(sentences, emb_table, W_ih, W_hh, b_ih, b_hh):
    batch, seq = sentences.shape
    vocab, emb = emb_table.shape
    hid = W_hh.shape[0]
    n_rows = batch * seq

    table2 = emb_table.reshape(vocab // 2, 2 * emb)

    sent_tm = sentences.T.astype(jnp.int32)
    phys = (sent_tm >> 1).reshape(n_rows)
    mask8 = jnp.broadcast_to(
        (sent_tm & 1).astype(jnp.int8)[:, :, None], (seq, batch, emb)
    )

    xg = _make_sc_gather(n_rows, 2 * emb, 32)(phys, table2)
    xg = xg.reshape(seq, batch, 2 * emb)

    wih_t = W_ih.T
    whh_t = W_hh.T
    bias = (b_ih + b_hh).reshape(1, hid)

    inner = 8
    n_blocks = (seq + inner - 1) // inner
    out_tm, h_last = pl.pallas_call(
        _make_rnn_body(inner, seq),
        grid=(n_blocks,),
        in_specs=[
            pl.BlockSpec((inner, batch, 2 * emb), lambda g: (g, 0, 0)),
            pl.BlockSpec((inner, batch, emb), lambda g: (g, 0, 0)),
            pl.BlockSpec((emb, hid), lambda g: (0, 0)),
            pl.BlockSpec((hid, hid), lambda g: (0, 0)),
            pl.BlockSpec((1, hid), lambda g: (0, 0)),
        ],
        out_specs=[
            pl.BlockSpec((inner, batch, hid), lambda g: (g, 0, 0)),
            pl.BlockSpec((batch, hid), lambda g: (0, 0)),
        ],
        out_shape=[
            jax.ShapeDtypeStruct((seq, batch, hid), jnp.float32),
            jax.ShapeDtypeStruct((batch, hid), jnp.float32),
        ],
        scratch_shapes=[pltpu.VMEM((batch, hid), jnp.float32)],
    )(xg, mask8, wih_t, whh_t, bias)

    return out_tm.transpose(1, 0, 2), h_last[None]


# speedup vs baseline: 1.2527x; 1.0724x over previous
import functools

import jax
import jax.numpy as jnp
from jax import lax
from jax.experimental import pallas as pl
from jax.experimental.pallas import tpu as pltpu
from jax.experimental.pallas import tpu_sc as plsc

_CHUNK = 80


def _prep_body(sent_ref, phys_ref):
    phys_ref[...] = (sent_ref[...] >> 1).T


def _make_sc_gather(n_rows, width, n_workers):
    rows_per_w = n_rows // n_workers
    n_chunks = rows_per_w // _CHUNK
    assert rows_per_w % _CHUNK == 0

    mesh = plsc.VectorSubcoreMesh(core_axis_name="c", subcore_axis_name="s")

    @functools.partial(
        pl.kernel,
        mesh=mesh,
        out_type=jax.ShapeDtypeStruct((n_rows, width), jnp.float32),
        scratch_types=[
            pltpu.VMEM((rows_per_w,), jnp.int32),
            pltpu.VMEM((2, _CHUNK, width), jnp.float32),
            pltpu.SemaphoreType.DMA,
        ],
    )
    def sc_gather(idx_hbm, table_hbm, out_hbm, idx_v, rows_v, sem):
        nc = 2
        wid = lax.axis_index("s") * nc + lax.axis_index("c")
        base = wid * rows_per_w
        pltpu.sync_copy(idx_hbm.at[pl.ds(base, rows_per_w)], idx_v)

        def gather(j, buf):
            return pltpu.async_copy(
                table_hbm.at[idx_v.at[pl.ds(j * _CHUNK, _CHUNK)]],
                rows_v.at[buf],
                sem,
            )

        cp = gather(0, 0)
        for j in range(n_chunks):
            nxt = gather(j + 1, (j + 1) % 2) if j + 1 < n_chunks else None
            cp.wait()
            pltpu.sync_copy(
                rows_v.at[j % 2], out_hbm.at[pl.ds(base + j * _CHUNK, _CHUNK)]
            )
            cp = nxt

    return sc_gather


def _make_rnn_body(inner, seq, batch):
    last_g, last_s = (seq - 1) // inner, (seq - 1) % inner

    def _rnn_steps(x_ref, par_ref, wih_ref, whh_ref, bias_ref, out_ref,
                   hout_ref, h_ref):
        g = pl.program_id(0)

        @pl.when(g == 0)
        def _():
            h_ref[...] = jnp.zeros_like(h_ref)

        hid = h_ref.shape[-1]
        iot = lax.broadcasted_iota(jnp.int32, (seq, 1), 0)
        h = h_ref[...]
        for s in range(inner):
            x128 = x_ref[pl.ds(s * batch, batch)]
            onehot = (iot == g * inner + s).astype(jnp.float32)
            par = jnp.dot(par_ref[...], onehot, preferred_element_type=jnp.float32)
            x = jnp.where(par > 0.5, x128[:, hid:], x128[:, :hid])
            acc = jnp.dot(x, wih_ref[...], preferred_element_type=jnp.float32)
            acc += jnp.dot(h, whh_ref[...], preferred_element_type=jnp.float32)
            acc += bias_ref[...]
            h = jnp.tanh(acc)
            out_ref[pl.ds(s * batch, batch)] = h
            if s == last_s:
                h_keep = h

        @pl.when(g == last_g)
        def _():
            hout_ref[...] = h_keep

        h_ref[...] = h

    return _rnn_steps


def kernel(sentences, emb_table, W_ih, W_hh, b_ih, b_hh):
    batch, seq = sentences.shape
    vocab, emb = emb_table.shape
    hid = W_hh.shape[0]
    n_rows = batch * seq

    table2 = emb_table.reshape(vocab // 2, 2 * emb)
    sent = sentences.astype(jnp.int32)
    parity = (sent & 1).astype(jnp.float32)

    phys_tm = pl.pallas_call(
        _prep_body,
        out_shape=jax.ShapeDtypeStruct((seq, batch), jnp.int32),
    )(sent)
    phys = phys_tm.reshape(n_rows)

    xg = _make_sc_gather(n_rows, 2 * emb, 32)(phys, table2)

    wih_t = W_ih.T
    whh_t = W_hh.T
    bias = (b_ih + b_hh).reshape(1, hid)

    inner = 8
    n_blocks = (seq + inner - 1) // inner
    out_tm, h_last = pl.pallas_call(
        _make_rnn_body(inner, seq, batch),
        grid=(n_blocks,),
        in_specs=[
            pl.BlockSpec((inner * batch, 2 * emb), lambda g: (g, 0)),
            pl.BlockSpec((batch, seq), lambda g: (0, 0)),
            pl.BlockSpec((emb, hid), lambda g: (0, 0)),
            pl.BlockSpec((hid, hid), lambda g: (0, 0)),
            pl.BlockSpec((1, hid), lambda g: (0, 0)),
        ],
        out_specs=[
            pl.BlockSpec((inner * batch, hid), lambda g: (g, 0)),
            pl.BlockSpec((batch, hid), lambda g: (0, 0)),
        ],
        out_shape=[
            jax.ShapeDtypeStruct((n_rows, hid), jnp.float32),
            jax.ShapeDtypeStruct((batch, hid), jnp.float32),
        ],
        scratch_shapes=[pltpu.VMEM((batch, hid), jnp.float32)],
    )(xg, parity, wih_t, whh_t, bias)

    final_output = out_tm.reshape(seq, batch, hid).transpose(1, 0, 2)
    return final_output, h_last[None]


# speedup vs baseline: 1.2588x; 1.0049x over previous
import functools

import jax
import jax.numpy as jnp
from jax import lax
from jax.experimental import pallas as pl
from jax.experimental.pallas import tpu as pltpu
from jax.experimental.pallas import tpu_sc as plsc

_CHUNK = 80


def _prep_body(sent_ref, idx_ref):
    idx_ref[...] = sent_ref[...].T


def _make_sc_gather(n_rows, width, n_workers):
    rows_per_w = n_rows // n_workers
    n_chunks = rows_per_w // _CHUNK
    assert rows_per_w % _CHUNK == 0

    mesh = plsc.VectorSubcoreMesh(core_axis_name="c", subcore_axis_name="s")

    @functools.partial(
        pl.kernel,
        mesh=mesh,
        out_type=jax.ShapeDtypeStruct((n_rows, width), jnp.float32),
        scratch_types=[
            pltpu.VMEM((rows_per_w,), jnp.int32),
            pltpu.VMEM((2, _CHUNK, width), jnp.float32),
            pltpu.SemaphoreType.DMA,
        ],
        compiler_params=pltpu.CompilerParams(use_tc_tiling_on_sc=False),
    )
    def sc_gather(idx_hbm, table_hbm, out_hbm, idx_v, rows_v, sem):
        nc = 2
        wid = lax.axis_index("s") * nc + lax.axis_index("c")
        base = wid * rows_per_w
        pltpu.sync_copy(idx_hbm.at[pl.ds(base, rows_per_w)], idx_v)

        def gather(j, buf):
            return pltpu.async_copy(
                table_hbm.at[idx_v.at[pl.ds(j * _CHUNK, _CHUNK)]],
                rows_v.at[buf],
                sem,
            )

        cp = gather(0, 0)
        for j in range(n_chunks):
            nxt = gather(j + 1, (j + 1) % 2) if j + 1 < n_chunks else None
            cp.wait()
            pltpu.sync_copy(
                rows_v.at[j % 2], out_hbm.at[pl.ds(base + j * _CHUNK, _CHUNK)]
            )
            cp = nxt

    return sc_gather


def _make_rnn_body(inner, seq, rows_t):
    last_g, last_s = (seq - 1) // inner, (seq - 1) % inner

    def _rnn_steps(x_ref, wih_ref, whh_ref, bias_ref, out_ref, hout_ref, h_ref):
        g = pl.program_id(0)

        @pl.when(g == 0)
        def _():
            h_ref[...] = jnp.zeros_like(h_ref)

        h = h_ref[...]
        for s in range(inner):
            x2 = x_ref[pl.ds(s * rows_t, rows_t)]
            acc = jnp.dot(x2, wih_ref[...], preferred_element_type=jnp.float32)
            acc += jnp.dot(h, whh_ref[...], preferred_element_type=jnp.float32)
            acc += bias_ref[...]
            h = jnp.tanh(acc)
            out_ref[pl.ds(s * rows_t, rows_t)] = h
            if s == last_s:
                h_keep = h

        @pl.when(g == last_g)
        def _():
            hout_ref[...] = h_keep

        h_ref[...] = h

    return _rnn_steps


def _blockdiag(w):
    n = w.shape[0]
    z = jnp.zeros((n, n), w.dtype)
    return jnp.block([[w, z], [z, w]])


def kernel(sentences, emb_table, W_ih, W_hh, b_ih, b_hh):
    batch, seq = sentences.shape
    vocab, emb = emb_table.shape
    hid = W_hh.shape[0]
    n_rows = batch * seq
    rows_t = batch // 2

    sent = sentences.astype(jnp.int32)

    idx_tm = pl.pallas_call(
        _prep_body,
        out_shape=jax.ShapeDtypeStruct((seq, batch), jnp.int32),
    )(sent)
    idx = idx_tm.reshape(n_rows)

    xg = _make_sc_gather(n_rows, emb, 32)(idx, emb_table)
    x2 = xg.reshape(n_rows // 2, 2 * emb)

    wih2 = _blockdiag(W_ih.T)
    whh2 = _blockdiag(W_hh.T)
    bias = b_ih + b_hh
    bias2 = jnp.concatenate([bias, bias]).reshape(1, 2 * hid)

    inner = 8
    n_blocks = (seq + inner - 1) // inner
    out2, h_last2 = pl.pallas_call(
        _make_rnn_body(inner, seq, rows_t),
        grid=(n_blocks,),
        in_specs=[
            pl.BlockSpec((inner * rows_t, 2 * emb), lambda g: (g, 0)),
            pl.BlockSpec((2 * emb, 2 * hid), lambda g: (0, 0)),
            pl.BlockSpec((2 * hid, 2 * hid), lambda g: (0, 0)),
            pl.BlockSpec((1, 2 * hid), lambda g: (0, 0)),
        ],
        out_specs=[
            pl.BlockSpec((inner * rows_t, 2 * hid), lambda g: (g, 0)),
            pl.BlockSpec((rows_t, 2 * hid), lambda g: (0, 0)),
        ],
        out_shape=[
            jax.ShapeDtypeStruct((n_rows // 2, 2 * hid), jnp.float32),
            jax.ShapeDtypeStruct((rows_t, 2 * hid), jnp.float32),
        ],
        scratch_shapes=[pltpu.VMEM((rows_t, 2 * hid), jnp.float32)],
    )(x2, wih2, whh2, bias2)

    final_output = out2.reshape(seq, batch, hid).transpose(1, 0, 2)
    h = h_last2.reshape(batch, hid)[None]
    return final_output, h
